# Initial kernel scaffold; baseline (speedup 1.0000x reference)
import jax
import jax.numpy as jnp
from jax.experimental import pallas as pl


def kernel(weight, token_ids):
    raise NotImplementedError("write your pallas kernel here")


# speedup vs baseline: 1.8647x; 1.8647x over previous
import functools

import jax
import jax.numpy as jnp
from jax import lax
from jax.experimental import pallas as pl
from jax.experimental.pallas import tpu as pltpu
from jax.experimental.pallas import tpu_sc as plsc

D = 64
NC, NS = 2, 16
NW = NC * NS


def _make_gather(B, C):
    b_per_w = B // NW
    nchunk = b_per_w // C
    mesh = plsc.VectorSubcoreMesh(core_axis_name="c", subcore_axis_name="s")

    @functools.partial(
        pl.kernel,
        out_type=jax.ShapeDtypeStruct((B, D), jnp.float32),
        mesh=mesh,
        scratch_types=[
            pltpu.VMEM((C,), jnp.int32),
            pltpu.VMEM((C, D), jnp.float32),
            pltpu.SemaphoreType.DMA,
        ],
        compiler_params=pltpu.CompilerParams(use_tc_tiling_on_sc=False),
    )
    def gather_kernel(table_hbm, idx_hbm, out_hbm, idx_v, rows_v, sem):
        wid = lax.axis_index("s") * NC + lax.axis_index("c")
        base = wid * b_per_w

        def body(i, carry):
            off = pl.multiple_of(base + i * C, 8)
            pltpu.sync_copy(idx_hbm.at[pl.ds(off, C)], idx_v)
            pltpu.async_copy(table_hbm.at[idx_v], rows_v, sem).wait()
            pltpu.sync_copy(rows_v, out_hbm.at[pl.ds(off, C)])
            return carry

        lax.fori_loop(0, nchunk, body, 0)

    return gather_kernel


def kernel(weight, token_ids):
    B0, S = token_ids.shape
    flat = token_ids.reshape(B0 * S)
    out = _make_gather(B0 * S, 1600)(weight, flat)
    return out.reshape(B0, S, D)


# speedup vs baseline: 1.8721x; 1.0040x over previous
import functools

import jax
import jax.numpy as jnp
from jax import lax
from jax.experimental import pallas as pl
from jax.experimental.pallas import tpu as pltpu
from jax.experimental.pallas import tpu_sc as plsc

D = 64
NC, NS = 2, 16
NW = NC * NS


def _make_gather(B, C):
    b_per_w = B // NW
    nchunk = b_per_w // C
    assert nchunk % 2 == 0 and nchunk >= 4
    mesh = plsc.VectorSubcoreMesh(core_axis_name="c", subcore_axis_name="s")

    @functools.partial(
        pl.kernel,
        out_type=jax.ShapeDtypeStruct((B, D), jnp.float32),
        mesh=mesh,
        scratch_types=[
            pltpu.VMEM((2, C), jnp.int32),
            pltpu.VMEM((2, C, D), jnp.float32),
            pltpu.SemaphoreType.DMA,
            pltpu.SemaphoreType.DMA,
            pltpu.SemaphoreType.DMA,
            pltpu.SemaphoreType.DMA,
            pltpu.SemaphoreType.DMA,
            pltpu.SemaphoreType.DMA,
        ],
        compiler_params=pltpu.CompilerParams(use_tc_tiling_on_sc=False),
    )
    def gather_kernel(table_hbm, idx_hbm, out_hbm, idx_v, rows_v,
                      si0, si1, sg0, sg1, sw0, sw1):
        s_idx = (si0, si1)
        s_g = (sg0, sg1)
        s_w = (sw0, sw1)
        wid = lax.axis_index("s") * NC + lax.axis_index("c")
        base = wid * b_per_w

        def chunk_off(k):
            return pl.multiple_of(base + k * C, 8)

        def pf_off(k):
            return pl.multiple_of(jnp.minimum(base + k * C, B - C), 8)

        def start_idx(k, b):
            pltpu.async_copy(idx_hbm.at[pl.ds(pf_off(k), C)],
                             idx_v.at[b], s_idx[b])

        def wait_idx(b):
            pltpu.make_async_copy(idx_hbm.at[pl.ds(0, C)],
                                  idx_v.at[b], s_idx[b]).wait()

        def start_g(b):
            pltpu.async_copy(table_hbm.at[idx_v.at[b]], rows_v.at[b], s_g[b])

        def wait_g(b):
            pltpu.make_async_copy(table_hbm.at[pl.ds(0, C)],
                                  rows_v.at[b], s_g[b]).wait()

        def start_w(k, b):
            pltpu.async_copy(rows_v.at[b],
                             out_hbm.at[pl.ds(chunk_off(k), C)], s_w[b])

        def wait_w(b):
            pltpu.make_async_copy(rows_v.at[b],
                                  out_hbm.at[pl.ds(0, C)], s_w[b]).wait()

        start_idx(0, 0)
        wait_idx(0)
        start_g(0)
        start_idx(1, 1)
        wait_g(0)
        wait_idx(1)
        start_g(1)
        start_w(0, 0)
        start_idx(2, 0)

        def lap(k, b):
            nb = 1 - b
            wait_g(b)
            wait_idx(nb)
            wait_w(nb)
            start_g(nb)
            start_w(k, b)
            start_idx(k + 2, b)

        def body(j, carry):
            lap(1 + 2 * j, 1)
            lap(2 + 2 * j, 0)
            return carry

        lax.fori_loop(0, (nchunk - 2) // 2, body, 0)

        wait_g(1)
        start_w(nchunk - 1, 1)
        wait_w(0)
        wait_w(1)
        wait_idx(0)

    return gather_kernel


def kernel(weight, token_ids):
    B0, S = token_ids.shape
    flat = token_ids.reshape(B0 * S)
    out = _make_gather(B0 * S, 800)(weight, flat)
    return out.reshape(B0, S, D)


# speedup vs baseline: 2.5224x; 1.3474x over previous
import functools

import jax
import jax.numpy as jnp
from jax import lax
from jax.experimental import pallas as pl
from jax.experimental.pallas import tpu as pltpu
from jax.experimental.pallas import tpu_sc as plsc

D = 64
S = 50
S_PAD = 56
NC, NS = 2, 16
NW = NC * NS


def _make_gather(n_b, nb_per_grp):
    b_per_w = n_b // NW
    ngrp = b_per_w // nb_per_grp
    C = nb_per_grp * S
    assert ngrp % 2 == 0 and ngrp >= 4
    mesh = plsc.VectorSubcoreMesh(core_axis_name="c", subcore_axis_name="s")

    @functools.partial(
        pl.kernel,
        out_type=jax.ShapeDtypeStruct((n_b, S_PAD, 128), jnp.float32),
        mesh=mesh,
        scratch_types=[
            pltpu.VMEM((2, C), jnp.int32),
            pltpu.VMEM((2, C, D), jnp.float32),
            pltpu.SemaphoreType.DMA,
            pltpu.SemaphoreType.DMA,
            pltpu.SemaphoreType.DMA,
            pltpu.SemaphoreType.DMA,
            pltpu.SemaphoreType.DMA,
            pltpu.SemaphoreType.DMA,
        ],
        compiler_params=pltpu.CompilerParams(use_tc_tiling_on_sc=False),
    )
    def gather_kernel(table_hbm, idx_hbm, out_hbm, idx_v, rows_v,
                      si0, si1, sg0, sg1, sw0, sw1):
        s_idx = (si0, si1)
        s_g = (sg0, sg1)
        s_w = (sw0, sw1)
        wid = lax.axis_index("s") * NC + lax.axis_index("c")
        base_b = wid * b_per_w

        def idx_off(k):
            return pl.multiple_of(
                jnp.minimum((base_b + k * nb_per_grp) * S, (n_b - nb_per_grp) * S), 8)

        def start_idx(k, b):
            pltpu.async_copy(idx_hbm.at[pl.ds(idx_off(k), C)],
                             idx_v.at[b], s_idx[b])

        def wait_idx(b):
            pltpu.make_async_copy(idx_hbm.at[pl.ds(0, C)],
                                  idx_v.at[b], s_idx[b]).wait()

        def start_g(b):
            pltpu.async_copy(table_hbm.at[idx_v.at[b]], rows_v.at[b], s_g[b])

        def wait_g(b):
            pltpu.make_async_copy(table_hbm.at[pl.ds(0, C)],
                                  rows_v.at[b], s_g[b]).wait()

        def start_w(k, b):
            b0 = base_b + k * nb_per_grp
            for j in range(nb_per_grp):
                pltpu.async_copy(
                    rows_v.at[b, pl.ds(j * S, S), :],
                    out_hbm.at[b0 + j, pl.ds(0, S), pl.ds(0, D)],
                    s_w[b])

        def wait_w(b):
            for _ in range(nb_per_grp):
                pltpu.make_async_copy(
                    rows_v.at[b, pl.ds(0, S), :],
                    out_hbm.at[0, pl.ds(0, S), pl.ds(0, D)],
                    s_w[b]).wait()

        start_idx(0, 0)
        wait_idx(0)
        start_g(0)
        start_idx(1, 1)
        wait_g(0)
        wait_idx(1)
        start_g(1)
        start_w(0, 0)
        start_idx(2, 0)

        def lap(k, b):
            nb = 1 - b
            wait_g(b)
            wait_idx(nb)
            wait_w(nb)
            start_g(nb)
            start_w(k, b)
            start_idx(k + 2, b)

        def body(j, carry):
            lap(1 + 2 * j, 1)
            lap(2 + 2 * j, 0)
            return carry

        lax.fori_loop(0, (ngrp - 2) // 2, body, 0)

        wait_g(1)
        start_w(ngrp - 1, 1)
        wait_w(0)
        wait_w(1)
        wait_idx(0)

    return gather_kernel


def kernel(weight, token_ids):
    B0, S0 = token_ids.shape
    flat = token_ids.reshape(B0 * S0)
    out_pad = _make_gather(B0, 16)(weight, flat)
    return out_pad[:, :S0, :D]
